# baseline (speedup 1.0000x reference)
import jax
import jax.numpy as jnp
from jax import lax
from jax.experimental import pallas as pl
from jax.experimental.pallas import tpu as pltpu
from jax.experimental.pallas import tpu_sc as plsc

_VOCAB = 100000
_EMB = 128
_SEP = 102
_B = 1024
_L = 200
_NC = 2
_NS = 16
_NW = _NC * _NS
_ROWS_W = _B // _NW
_TOK_W = _ROWS_W * _L
_UT = 128
_NU = _TOK_W // _UT
_BIG = 1 << 30


def _positional_embedding():
    pos = jnp.arange(_L, dtype=jnp.float32)[:, None]
    i = jnp.arange(_EMB)[None, :]
    angle = pos / jnp.power(10000.0, (2.0 * (i // 2)).astype(jnp.float32) / _EMB)
    return jnp.where(i % 2 == 0, jnp.sin(angle), jnp.cos(angle))


def _body(inp_hbm, word_hbm, seg_hbm, pe_hbm, out_hbm,
          idx_all, c0_v, n1_v, ch0, ch1, ch2, ch3,
          sg0, sg1, sg2, sg3, sw0, sw1, sw2, sw3):
    wid = lax.axis_index("s") * _NC + lax.axis_index("c")
    tbase = wid * _TOK_W

    pltpu.sync_copy(inp_hbm.at[pl.ds(pl.multiple_of(tbase, 8), _TOK_W)], idx_all)

    pltpu.sync_copy(seg_hbm, ch1.at[pl.ds(0, 2)])
    s0 = [ch1[0, pl.ds(k * 16, 16)] for k in range(8)]
    s1 = [ch1[1, pl.ds(k * 16, 16)] for k in range(8)]
    delta = [s1[k] - s0[k] for k in range(8)]

    def rne16(x):
        u = lax.bitcast_convert_type(x, jnp.int32)
        r = u + jnp.int32(0x7FFF) + (lax.shift_right_logical(u, 16) & 1)
        return lax.shift_right_logical(r, 16)

    def build_rows(base, count):
        def build(r, _):
            for k in range(4):
                a = ch0[r, pl.ds(k * 32, 16)] + s0[2 * k]
                b2 = ch0[r, pl.ds(k * 32 + 16, 16)] + s0[2 * k + 1]
                c0_v[r + base, pl.ds(k * 16, 16)] = (
                    rne16(a) | lax.shift_left(rne16(b2), 16))
            return 0
        lax.fori_loop(0, count, build, 0)

    pltpu.sync_copy(pe_hbm.at[pl.ds(0, _UT)], ch0)
    build_rows(0, _UT)
    pltpu.sync_copy(pe_hbm.at[pl.ds(_UT, _L - _UT)], ch0.at[pl.ds(0, _L - _UT)])
    build_rows(_UT, _L - _UT)

    def scan_row(r, _):
        off = r * _L
        rm = jnp.full((16,), _BIG, jnp.int32)
        for j in range(13):
            o = min(j * 16, _L - 16)
            v = idx_all[pl.ds(off + o, 16)]
            posv = lax.iota(jnp.int32, 16) + o
            rm = jnp.minimum(rm, jnp.where(v == _SEP, posv, _BIG))
        m = rm[0]
        for j in range(1, 16):
            m = jnp.minimum(m, rm[j])
        n1 = jnp.where(m >= _BIG, jnp.int32(0), m + 1)
        n1_v[r, pl.ds(0, 16)] = jnp.full((16,), 0, jnp.int32) + n1
        return 0

    lax.fori_loop(0, _ROWS_W, scan_row, 0)

    chunks = (ch0, ch1, ch2, ch3)
    sgs = (sg0, sg1, sg2, sg3)
    sws = (sw0, sw1, sw2, sw3)

    def fire_gather(u, s):
        off = pl.multiple_of(u * _UT, 8)
        pltpu.async_copy(word_hbm.at[idx_all.at[pl.ds(off, _UT)]],
                         chunks[s], sgs[s])

    def wait_gather(s):
        pltpu.make_async_copy(word_hbm.at[idx_all.at[pl.ds(0, _UT)]],
                              chunks[s], sgs[s]).wait()

    def fire_write(u, s):
        dst = pl.multiple_of(tbase + u * _UT, 8)
        pltpu.async_copy(chunks[s], out_hbm.at[pl.ds(dst, _UT)], sws[s])

    def wait_write(s):
        pltpu.make_async_copy(chunks[s], out_hbm.at[pl.ds(0, _UT)], sws[s]).wait()

    def unpk(w):
        a = lax.bitcast_convert_type(lax.shift_left(w, 16), jnp.float32)
        b2 = lax.bitcast_convert_type(w & jnp.int32(-65536), jnp.float32)
        return a, b2

    def compute(u, s):
        ch = chunks[s]
        a0 = u * _UT
        r0 = a0 // _L
        l0 = a0 - r0 * _L
        m1 = jnp.minimum(_L - l0, _UT)
        n1a = n1_v[r0, pl.ds(0, 16)][0]
        r1 = jnp.minimum(r0 + 1, _ROWS_W - 1)
        n1b = n1_v[r1, pl.ds(0, 16)][0]
        split1 = jnp.clip(n1a - l0, 0, m1)
        split2 = m1 + jnp.clip(n1b, 0, _UT - m1)

        def add_range(lo, hi, coff, with_delta):
            @plsc.parallel_loop(lo, hi, unroll=4)
            def tok(t):
                crow = t + coff
                for k in range(4):
                    av, bv = unpk(c0_v[crow, pl.ds(k * 16, 16)])
                    sla = pl.ds(k * 32, 16)
                    slb = pl.ds(k * 32 + 16, 16)
                    if with_delta:
                        ch[t, sla] = ch[t, sla] + av + delta[2 * k]
                        ch[t, slb] = ch[t, slb] + bv + delta[2 * k + 1]
                    else:
                        ch[t, sla] = ch[t, sla] + av
                        ch[t, slb] = ch[t, slb] + bv

        add_range(0, split1, l0, True)
        add_range(split1, m1, l0, False)
        add_range(m1, split2, -m1, True)
        add_range(split2, _UT, -m1, False)

    fire_gather(0, 0)
    fire_gather(1, 1)
    fire_gather(2, 2)

    def grp(g, _):
        for b in range(4):
            u = 4 * g + b

            @pl.when(u < _NU)
            def _():
                wait_gather(b)
                compute(u, b)
                fire_write(u, b)

            s2 = (b + 3) % 4
            v = u + 3

            @pl.when(v == 3)
            def _():
                fire_gather(3, 3)

            @pl.when((v >= 4) & (v < _NU))
            def _():
                wait_write(s2)
                fire_gather(v, s2)

        return 0

    lax.fori_loop(0, (_NU + 5) // 4, grp, 0)

    wait_write(2); wait_write(3); wait_write(0); wait_write(1)


@jax.jit
def _run(inp_flat, word_table, seg_table, pe):
    mesh = plsc.VectorSubcoreMesh(core_axis_name="c", subcore_axis_name="s")
    return pl.kernel(
        _body,
        out_type=jax.ShapeDtypeStruct((_B * _L, _EMB), jnp.float32),
        mesh=mesh,
        scratch_types=[
            pltpu.VMEM((_TOK_W,), jnp.int32),
            pltpu.VMEM((_L, _EMB // 2), jnp.int32),
            pltpu.VMEM((_ROWS_W, 16), jnp.int32),
            pltpu.VMEM((_UT, _EMB), jnp.float32),
            pltpu.VMEM((_UT, _EMB), jnp.float32),
            pltpu.VMEM((_UT, _EMB), jnp.float32),
            pltpu.VMEM((_UT, _EMB), jnp.float32),
            pltpu.SemaphoreType.DMA,
            pltpu.SemaphoreType.DMA,
            pltpu.SemaphoreType.DMA,
            pltpu.SemaphoreType.DMA,
            pltpu.SemaphoreType.DMA,
            pltpu.SemaphoreType.DMA,
            pltpu.SemaphoreType.DMA,
            pltpu.SemaphoreType.DMA,
        ],
    )(inp_flat, word_table, seg_table, pe)


def kernel(inp, word_table, seg_table):
    inp_flat = inp.reshape(-1).astype(jnp.int32)
    pe = _positional_embedding()
    out = _run(inp_flat, word_table, seg_table, pe)
    return out.reshape(_B, _L, _EMB)


# speedup vs baseline: 1.1141x; 1.1141x over previous
import jax
import jax.numpy as jnp
from jax import lax
from jax.experimental import pallas as pl
from jax.experimental.pallas import tpu as pltpu
from jax.experimental.pallas import tpu_sc as plsc

_VOCAB = 100000
_EMB = 128
_SEP = 102
_B = 1024
_L = 200
_NC = 2
_NS = 16
_NW = _NC * _NS
_ROWS_W = _B // _NW
_BIG = 1 << 30


def _positional_embedding():
    pos = jnp.arange(_L, dtype=jnp.float32)[:, None]
    i = jnp.arange(_EMB)[None, :]
    angle = pos / jnp.power(10000.0, (2.0 * (i // 2)).astype(jnp.float32) / _EMB)
    return jnp.where(i % 2 == 0, jnp.sin(angle), jnp.cos(angle))


def _body(inp_hbm, word_hbm, seg_hbm, pe_hbm, out_hbm,
          idx_all, c0_v, ch0, ch1, ch2, segb_v,
          sg0, sg1, sg2, sw0, sw1, sw2):
    wid = lax.axis_index("s") * _NC + lax.axis_index("c")
    w0 = wid * _ROWS_W

    pltpu.sync_copy(
        inp_hbm.at[pl.ds(pl.multiple_of(w0 * _L, 8), _ROWS_W * _L)], idx_all)

    pltpu.sync_copy(pe_hbm, ch0)
    pltpu.sync_copy(seg_hbm, segb_v)
    s0 = [segb_v[0, pl.ds(k * 16, 16)] for k in range(8)]
    s1 = [segb_v[1, pl.ds(k * 16, 16)] for k in range(8)]
    delta = [s1[k] - s0[k] for k in range(8)]

    def rne16(x):
        u = plsc.bitcast(x, jnp.int32)
        r = u + jnp.int32(0x7FFF) + (lax.shift_right_logical(u, 16) & 1)
        return lax.shift_right_logical(r, 16)

    def build_c0(r, _):
        for k in range(4):
            a = ch0[r, pl.ds(k * 32, 16)] + s0[2 * k]
            b2 = ch0[r, pl.ds(k * 32 + 16, 16)] + s0[2 * k + 1]
            c0_v[r, pl.ds(k * 16, 16)] = (
                rne16(a) | lax.shift_left(rne16(b2), 16))
        return 0

    lax.fori_loop(0, _L, build_c0, 0)

    chunks = (ch0, ch1, ch2)
    sgs = (sg0, sg1, sg2)
    sws = (sw0, sw1, sw2)

    def fire_gather(i, s):
        off = pl.multiple_of(i * _L, 8)
        pltpu.async_copy(word_hbm.at[idx_all.at[pl.ds(off, 104)]],
                         chunks[s].at[pl.ds(0, 104)], sgs[s])
        pltpu.async_copy(word_hbm.at[idx_all.at[pl.ds(off + 104, 96)]],
                         chunks[s].at[pl.ds(104, 96)], sgs[s])

    def wait_gather(s):
        pltpu.make_async_copy(word_hbm.at[idx_all.at[pl.ds(0, 104)]],
                              chunks[s].at[pl.ds(0, 104)], sgs[s]).wait()
        pltpu.make_async_copy(word_hbm.at[idx_all.at[pl.ds(104, 96)]],
                              chunks[s].at[pl.ds(104, 96)], sgs[s]).wait()

    def fire_write(i, s):
        pltpu.async_copy(chunks[s], out_hbm.at[w0 + i], sws[s])

    def wait_write(s):
        pltpu.make_async_copy(chunks[s], out_hbm.at[0], sws[s]).wait()

    def compute(i, s):
        off = i * _L
        rm = jnp.full((16,), _BIG, jnp.int32)
        for j in range(13):
            o = min(j * 16, _L - 16)
            v = idx_all[pl.ds(off + o, 16)]
            posv = lax.iota(jnp.int32, 16) + o
            rm = jnp.minimum(rm, jnp.where(v == _SEP, posv, _BIG))
        m = rm[0]
        for j in range(1, 16):
            m = jnp.minimum(m, rm[j])
        n1 = jnp.where(m >= _BIG, jnp.int32(0), m + 1)

        ch = chunks[s]

        def unpk(w):
            a = plsc.bitcast(lax.shift_left(w, 16), jnp.float32)
            b2 = plsc.bitcast(w & jnp.int32(-65536), jnp.float32)
            return a, b2

        @plsc.parallel_loop(0, n1, unroll=6)
        def tok1(t):
            for k in range(4):
                a, b2 = unpk(c0_v[t, pl.ds(k * 16, 16)])
                sla = pl.ds(k * 32, 16)
                slb = pl.ds(k * 32 + 16, 16)
                ch[t, sla] = ch[t, sla] + a + delta[2 * k]
                ch[t, slb] = ch[t, slb] + b2 + delta[2 * k + 1]

        @plsc.parallel_loop(n1, _L, unroll=6)
        def tok0(t):
            for k in range(4):
                a, b2 = unpk(c0_v[t, pl.ds(k * 16, 16)])
                sla = pl.ds(k * 32, 16)
                slb = pl.ds(k * 32 + 16, 16)
                ch[t, sla] = ch[t, sla] + a
                ch[t, slb] = ch[t, slb] + b2

    fire_gather(0, 0)
    fire_gather(1, 1)

    wait_gather(0); compute(0, 0); fire_write(0, 0)
    fire_gather(2, 2)
    wait_gather(1); compute(1, 1); fire_write(1, 1)
    wait_write(0); fire_gather(3, 0)
    wait_gather(2); compute(2, 2); fire_write(2, 2)
    wait_write(1); fire_gather(4, 1)

    def grp(g, _):
        for b in range(3):
            i = 3 * g + b
            wait_gather(b)
            compute(i, b)
            fire_write(i, b)
            s2 = (b + 2) % 3
            wait_write(s2)
            fire_gather(i + 2, s2)
        return 0

    lax.fori_loop(1, 10, grp, 0)

    wait_gather(0); compute(30, 0); fire_write(30, 0)
    wait_gather(1); compute(31, 1); fire_write(31, 1)
    wait_write(2); wait_write(0); wait_write(1)


@jax.jit
def _run(inp_flat, word_table, seg_table, pe):
    mesh = plsc.VectorSubcoreMesh(core_axis_name="c", subcore_axis_name="s")
    return pl.kernel(
        _body,
        out_type=jax.ShapeDtypeStruct((_B, _L, _EMB), jnp.float32),
        mesh=mesh,
        scratch_types=[
            pltpu.VMEM((_ROWS_W * _L,), jnp.int32),
            pltpu.VMEM((_L, _EMB // 2), jnp.int32),
            pltpu.VMEM((_L, _EMB), jnp.float32),
            pltpu.VMEM((_L, _EMB), jnp.float32),
            pltpu.VMEM((_L, _EMB), jnp.float32),
            pltpu.VMEM((2, _EMB), jnp.float32),
            pltpu.SemaphoreType.DMA,
            pltpu.SemaphoreType.DMA,
            pltpu.SemaphoreType.DMA,
            pltpu.SemaphoreType.DMA,
            pltpu.SemaphoreType.DMA,
            pltpu.SemaphoreType.DMA,
        ],
    )(inp_flat, word_table, seg_table, pe)


def kernel(inp, word_table, seg_table):
    inp_flat = inp.reshape(-1).astype(jnp.int32)
    pe = _positional_embedding()
    return _run(inp_flat, word_table, seg_table, pe)


# speedup vs baseline: 1.1198x; 1.0051x over previous
import jax
import jax.numpy as jnp
from jax import lax
from jax.experimental import pallas as pl
from jax.experimental.pallas import tpu as pltpu
from jax.experimental.pallas import tpu_sc as plsc

_VOCAB = 100000
_EMB = 128
_SEP = 102
_B = 1024
_L = 200
_NC = 2
_NS = 16
_NW = _NC * _NS
_ROWS_W = _B // _NW
_BIG = 1 << 30


def _positional_embedding():
    pos = jnp.arange(_L, dtype=jnp.float32)[:, None]
    i = jnp.arange(_EMB)[None, :]
    angle = pos / jnp.power(10000.0, (2.0 * (i // 2)).astype(jnp.float32) / _EMB)
    return jnp.where(i % 2 == 0, jnp.sin(angle), jnp.cos(angle))


def _body(inp_hbm, word_hbm, seg_hbm, pe_hbm, out_hbm,
          idx_all, c0_v, ch0, ch1, ch2, segb_v,
          sg0, sg1, sg2, sw0, sw1, sw2):
    wid = lax.axis_index("s") * _NC + lax.axis_index("c")
    w0 = wid * _ROWS_W

    pltpu.sync_copy(
        inp_hbm.at[pl.ds(pl.multiple_of(w0 * _L, 8), _ROWS_W * _L)], idx_all)

    pltpu.sync_copy(pe_hbm, ch0)
    pltpu.sync_copy(seg_hbm, segb_v)
    s0 = [segb_v[0, pl.ds(k * 16, 16)] for k in range(8)]
    s1 = [segb_v[1, pl.ds(k * 16, 16)] for k in range(8)]
    delta = [s1[k] - s0[k] for k in range(8)]

    def rne16(x):
        u = plsc.bitcast(x, jnp.int32)
        r = u + jnp.int32(0x7FFF) + (lax.shift_right_logical(u, 16) & 1)
        return lax.shift_right_logical(r, 16)

    def build_c0(r, _):
        for k in range(4):
            a = ch0[r, pl.ds(k * 32, 16)] + s0[2 * k]
            b2 = ch0[r, pl.ds(k * 32 + 16, 16)] + s0[2 * k + 1]
            c0_v[r, pl.ds(k * 16, 16)] = (
                rne16(a) | lax.shift_left(rne16(b2), 16))
        return 0

    lax.fori_loop(0, _L, build_c0, 0)

    chunks = (ch0, ch1, ch2)
    sgs = (sg0, sg1, sg2)
    sws = (sw0, sw1, sw2)

    def fire_gather(i, s):
        off = pl.multiple_of(i * _L, 8)
        pltpu.async_copy(word_hbm.at[idx_all.at[pl.ds(off, 104)]],
                         chunks[s].at[pl.ds(0, 104)], sgs[s])
        pltpu.async_copy(word_hbm.at[idx_all.at[pl.ds(off + 104, 96)]],
                         chunks[s].at[pl.ds(104, 96)], sgs[s])

    def wait_gather(s):
        pltpu.make_async_copy(word_hbm.at[idx_all.at[pl.ds(0, 104)]],
                              chunks[s].at[pl.ds(0, 104)], sgs[s]).wait()
        pltpu.make_async_copy(word_hbm.at[idx_all.at[pl.ds(104, 96)]],
                              chunks[s].at[pl.ds(104, 96)], sgs[s]).wait()

    def fire_write(i, s):
        pltpu.async_copy(chunks[s], out_hbm.at[w0 + i], sws[s])

    def wait_write(s):
        pltpu.make_async_copy(chunks[s], out_hbm.at[0], sws[s]).wait()

    def compute(i, s):
        off = i * _L
        rm = jnp.full((16,), _BIG, jnp.int32)
        for j in range(13):
            o = min(j * 16, _L - 16)
            v = idx_all[pl.ds(off + o, 16)]
            posv = lax.iota(jnp.int32, 16) + o
            rm = jnp.minimum(rm, jnp.where(v == _SEP, posv, _BIG))
        m = rm[0]
        for j in range(1, 16):
            m = jnp.minimum(m, rm[j])
        n1 = jnp.where(m >= _BIG, jnp.int32(0), m + 1)

        ch = chunks[s]

        def unpk(w):
            a = plsc.bitcast(lax.shift_left(w, 16), jnp.float32)
            b2 = plsc.bitcast(w & jnp.int32(-65536), jnp.float32)
            return a, b2

        @plsc.parallel_loop(0, n1, unroll=4)
        def tok1(t):
            for k in range(4):
                a, b2 = unpk(c0_v[t, pl.ds(k * 16, 16)])
                sla = pl.ds(k * 32, 16)
                slb = pl.ds(k * 32 + 16, 16)
                ch[t, sla] = ch[t, sla] + a + delta[2 * k]
                ch[t, slb] = ch[t, slb] + b2 + delta[2 * k + 1]

        @plsc.parallel_loop(n1, _L, unroll=4)
        def tok0(t):
            for k in range(4):
                a, b2 = unpk(c0_v[t, pl.ds(k * 16, 16)])
                sla = pl.ds(k * 32, 16)
                slb = pl.ds(k * 32 + 16, 16)
                ch[t, sla] = ch[t, sla] + a
                ch[t, slb] = ch[t, slb] + b2

    fire_gather(0, 0)
    fire_gather(1, 1)

    wait_gather(0); compute(0, 0); fire_write(0, 0)
    fire_gather(2, 2)
    wait_gather(1); compute(1, 1); fire_write(1, 1)
    wait_write(0); fire_gather(3, 0)
    wait_gather(2); compute(2, 2); fire_write(2, 2)
    wait_write(1); fire_gather(4, 1)

    def grp(g, _):
        for b in range(3):
            i = 3 * g + b
            wait_gather(b)
            compute(i, b)
            fire_write(i, b)
            s2 = (b + 2) % 3
            wait_write(s2)
            fire_gather(i + 2, s2)
        return 0

    lax.fori_loop(1, 10, grp, 0)

    wait_gather(0); compute(30, 0); fire_write(30, 0)
    wait_gather(1); compute(31, 1); fire_write(31, 1)
    wait_write(2); wait_write(0); wait_write(1)


@jax.jit
def _run(inp_flat, word_table, seg_table, pe):
    mesh = plsc.VectorSubcoreMesh(core_axis_name="c", subcore_axis_name="s")
    return pl.kernel(
        _body,
        out_type=jax.ShapeDtypeStruct((_B, _L, _EMB), jnp.float32),
        mesh=mesh,
        scratch_types=[
            pltpu.VMEM((_ROWS_W * _L,), jnp.int32),
            pltpu.VMEM((_L, _EMB // 2), jnp.int32),
            pltpu.VMEM((_L, _EMB), jnp.float32),
            pltpu.VMEM((_L, _EMB), jnp.float32),
            pltpu.VMEM((_L, _EMB), jnp.float32),
            pltpu.VMEM((2, _EMB), jnp.float32),
            pltpu.SemaphoreType.DMA,
            pltpu.SemaphoreType.DMA,
            pltpu.SemaphoreType.DMA,
            pltpu.SemaphoreType.DMA,
            pltpu.SemaphoreType.DMA,
            pltpu.SemaphoreType.DMA,
        ],
    )(inp_flat, word_table, seg_table, pe)


def kernel(inp, word_table, seg_table):
    inp_flat = inp.reshape(-1).astype(jnp.int32)
    pe = _positional_embedding()
    return _run(inp_flat, word_table, seg_table, pe)


# speedup vs baseline: 1.1303x; 1.0093x over previous
import jax
import jax.numpy as jnp
from jax import lax
from jax.experimental import pallas as pl
from jax.experimental.pallas import tpu as pltpu
from jax.experimental.pallas import tpu_sc as plsc

_VOCAB = 100000
_EMB = 128
_SEP = 102
_B = 1024
_L = 200
_NC = 2
_NS = 16
_NW = _NC * _NS
_ROWS_W = _B // _NW
_BIG = 1 << 30


def _positional_embedding():
    pos = jnp.arange(_L, dtype=jnp.float32)[:, None]
    i = jnp.arange(_EMB)[None, :]
    angle = pos / jnp.power(10000.0, (2.0 * (i // 2)).astype(jnp.float32) / _EMB)
    return jnp.where(i % 2 == 0, jnp.sin(angle), jnp.cos(angle))


def _body(inp_hbm, word_hbm, seg_hbm, pe_hbm, out_hbm,
          idx_all, c0_v, ch0, ch1, ch2, ch3,
          sg0, sg1, sg2, sg3, sw0, sw1, sw2, sw3):
    wid = lax.axis_index("s") * _NC + lax.axis_index("c")
    w0 = wid * _ROWS_W

    pltpu.sync_copy(
        inp_hbm.at[pl.ds(pl.multiple_of(w0 * _L, 8), 8 * _L)], idx_all)

    def idx_off(i):
        return pl.multiple_of(((i // 4) % 2) * (4 * _L) + (i % 4) * _L, 8)

    def refill_idx(i):
        pltpu.sync_copy(
            inp_hbm.at[pl.ds(pl.multiple_of((w0 + i + 5) * _L, 8), 4 * _L)],
            idx_all.at[pl.ds(((i // 4) % 2) * (4 * _L), 4 * _L)])

    pltpu.sync_copy(pe_hbm, ch0)
    pltpu.sync_copy(seg_hbm, ch1.at[pl.ds(0, 2)])
    s0 = [ch1[0, pl.ds(k * 16, 16)] for k in range(8)]
    s1 = [ch1[1, pl.ds(k * 16, 16)] for k in range(8)]
    delta = [s1[k] - s0[k] for k in range(8)]

    def rne16(x):
        u = lax.bitcast_convert_type(x, jnp.int32)
        r = u + jnp.int32(0x7FFF) + (lax.shift_right_logical(u, 16) & 1)
        return lax.shift_right_logical(r, 16)

    def build_c0(r, _):
        for k in range(4):
            a = ch0[r, pl.ds(k * 32, 16)] + s0[2 * k]
            b2 = ch0[r, pl.ds(k * 32 + 16, 16)] + s0[2 * k + 1]
            c0_v[r, pl.ds(k * 16, 16)] = (
                rne16(a) | lax.shift_left(rne16(b2), 16))
        return 0

    lax.fori_loop(0, _L, build_c0, 0)

    chunks = (ch0, ch1, ch2, ch3)
    sgs = (sg0, sg1, sg2, sg3)
    sws = (sw0, sw1, sw2, sw3)

    def fire_gather(i, s):
        off = idx_off(i)
        pltpu.async_copy(word_hbm.at[idx_all.at[pl.ds(off, 104)]],
                         chunks[s].at[pl.ds(0, 104)], sgs[s])
        pltpu.async_copy(word_hbm.at[idx_all.at[pl.ds(off + 104, 96)]],
                         chunks[s].at[pl.ds(104, 96)], sgs[s])

    def wait_gather(s):
        pltpu.make_async_copy(word_hbm.at[idx_all.at[pl.ds(0, 104)]],
                              chunks[s].at[pl.ds(0, 104)], sgs[s]).wait()
        pltpu.make_async_copy(word_hbm.at[idx_all.at[pl.ds(104, 96)]],
                              chunks[s].at[pl.ds(104, 96)], sgs[s]).wait()

    def fire_write(i, s):
        pltpu.async_copy(chunks[s], out_hbm.at[w0 + i], sws[s])

    def wait_write(s):
        pltpu.make_async_copy(chunks[s], out_hbm.at[0], sws[s]).wait()

    def compute(i, s):
        off = idx_off(i)
        rm = jnp.full((16,), _BIG, jnp.int32)
        for j in range(13):
            o = min(j * 16, _L - 16)
            v = idx_all[pl.ds(off + o, 16)]
            posv = lax.iota(jnp.int32, 16) + o
            rm = jnp.minimum(rm, jnp.where(v == _SEP, posv, _BIG))
        m = rm[0]
        for j in range(1, 16):
            m = jnp.minimum(m, rm[j])
        n1 = jnp.where(m >= _BIG, jnp.int32(0), m + 1)

        ch = chunks[s]

        def unpk(w):
            a = lax.bitcast_convert_type(lax.shift_left(w, 16), jnp.float32)
            b2 = lax.bitcast_convert_type(w & jnp.int32(-65536), jnp.float32)
            return a, b2

        @plsc.parallel_loop(0, n1, unroll=4)
        def tok1(t):
            for k in range(4):
                a, b2 = unpk(c0_v[t, pl.ds(k * 16, 16)])
                sla = pl.ds(k * 32, 16)
                slb = pl.ds(k * 32 + 16, 16)
                ch[t, sla] = ch[t, sla] + a + delta[2 * k]
                ch[t, slb] = ch[t, slb] + b2 + delta[2 * k + 1]

        @plsc.parallel_loop(n1, _L, unroll=4)
        def tok0(t):
            for k in range(4):
                a, b2 = unpk(c0_v[t, pl.ds(k * 16, 16)])
                sla = pl.ds(k * 32, 16)
                slb = pl.ds(k * 32 + 16, 16)
                ch[t, sla] = ch[t, sla] + a
                ch[t, slb] = ch[t, slb] + b2

    fire_gather(0, 0)
    fire_gather(1, 1)
    fire_gather(2, 2)

    wait_gather(0); compute(0, 0); fire_write(0, 0)
    fire_gather(3, 3)
    wait_gather(1); compute(1, 1); fire_write(1, 1)
    wait_write(0); fire_gather(4, 0)
    wait_gather(2); compute(2, 2); fire_write(2, 2)
    wait_write(1); fire_gather(5, 1)
    wait_gather(3); compute(3, 3); fire_write(3, 3)
    refill_idx(3)
    wait_write(2); fire_gather(6, 2)

    def grp(g, _):
        for b in range(4):
            i = 4 * g + b
            wait_gather(b)
            compute(i, b)
            fire_write(i, b)

            if b == 3:
                @pl.when(i <= _ROWS_W - 9)
                def _():
                    refill_idx(i)

            s2 = (b + 3) % 4

            @pl.when(i <= _ROWS_W - 4)
            def _():
                wait_write(s2)
                fire_gather(i + 3, s2)

        return 0

    lax.fori_loop(1, _ROWS_W // 4, grp, 0)

    wait_write(0); wait_write(1); wait_write(2); wait_write(3)


@jax.jit
def _run(inp_flat, word_table, seg_table, pe):
    mesh = plsc.VectorSubcoreMesh(core_axis_name="c", subcore_axis_name="s")
    return pl.kernel(
        _body,
        out_type=jax.ShapeDtypeStruct((_B, _L, _EMB), jnp.float32),
        mesh=mesh,
        scratch_types=[
            pltpu.VMEM((8 * _L,), jnp.int32),
            pltpu.VMEM((_L, _EMB // 2), jnp.int32),
            pltpu.VMEM((_L, _EMB), jnp.float32),
            pltpu.VMEM((_L, _EMB), jnp.float32),
            pltpu.VMEM((_L, _EMB), jnp.float32),
            pltpu.VMEM((_L, _EMB), jnp.float32),
            pltpu.SemaphoreType.DMA,
            pltpu.SemaphoreType.DMA,
            pltpu.SemaphoreType.DMA,
            pltpu.SemaphoreType.DMA,
            pltpu.SemaphoreType.DMA,
            pltpu.SemaphoreType.DMA,
            pltpu.SemaphoreType.DMA,
            pltpu.SemaphoreType.DMA,
        ],
    )(inp_flat, word_table, seg_table, pe)


def kernel(inp, word_table, seg_table):
    inp_flat = inp.reshape(-1).astype(jnp.int32)
    pe = _positional_embedding()
    return _run(inp_flat, word_table, seg_table, pe)
